# baseline (speedup 1.0000x reference)
import functools

import jax
import jax.numpy as jnp
from jax import lax
from jax.experimental import pallas as pl
from jax.experimental.pallas import tpu as pltpu
from jax.experimental.pallas import tpu_sc as plsc


def kernel(input_seqs, pos_emb):
    B, L = input_seqs.shape
    Lk, E = pos_emb.shape
    D = Lk * E
    flat = pos_emb.reshape(D)

    info = plsc.get_sparse_core_info()
    NC, NS = info.num_cores, info.num_subcores
    NW = NC * NS
    bpw = B // NW
    K = 16
    n_dma = bpw // K
    WINDOW = 8

    mesh = plsc.VectorSubcoreMesh(core_axis_name="c", subcore_axis_name="s")

    @functools.partial(
        pl.kernel,
        mesh=mesh,
        out_type=jax.ShapeDtypeStruct((B, D), jnp.float32),
        scratch_types=[
            pltpu.VMEM((K, D), jnp.float32),
            pltpu.SemaphoreType.DMA,
            pltpu.SemaphoreType.DMA,
        ],
    )
    def k(emb_hbm, out_hbm, buf, fill_sem, out_sem):
        wid = lax.axis_index("s") * NC + lax.axis_index("c")
        base = wid * bpw

        fills = [pltpu.async_copy(emb_hbm, buf.at[i], fill_sem) for i in range(K)]
        for c in fills:
            c.wait()

        pending = []
        for i in range(n_dma):
            if len(pending) == WINDOW:
                pending.pop(0).wait()
            pending.append(
                pltpu.async_copy(buf, out_hbm.at[pl.ds(base + i * K, K)], out_sem)
            )
        for c in pending:
            c.wait()

    out = k(flat)
    return out.reshape(B, L, E)
